# baseline (speedup 1.0000x reference)
import functools

import jax
import jax.numpy as jnp
from jax import lax
from jax.experimental import pallas as pl
from jax.experimental.pallas import tpu as pltpu
from jax.experimental.pallas import tpu_sc as plsc

_N = 10000
_E = 320000
_NC = 2
_NS = 16
_B = 128
_NP = 10112
_RPS = _NP // _NS
_EP = 327680
_EBLK = _EP // _B
_DH = 64
_K = 5


def _make_deg_kernel():
    mesh = plsc.VectorSubcoreMesh(core_axis_name="c", subcore_axis_name="s")
    nblk = _EBLK // (_NC * _NS)

    @functools.partial(
        pl.kernel,
        out_type=jax.ShapeDtypeStruct((_NC * _NP, 16), jnp.float32),
        mesh=mesh,
        scratch_types=[
            pltpu.VMEM((nblk, _B), jnp.int32),
            pltpu.VMEM((_B, 16), jnp.float32),
            pltpu.VMEM_SHARED((_NP, 16), jnp.float32),
        ],
        compiler_params=pltpu.CompilerParams(use_tc_tiling_on_sc=False),
    )
    def deg_kernel(dst_hbm, ones_hbm, zeros_hbm, out_hbm, dstv, onesv, acc):
        c = lax.axis_index("c")
        s = lax.axis_index("s")
        w = c * _NS + s
        pltpu.sync_copy(dst_hbm.at[pl.ds(w * nblk, nblk)], dstv)
        pltpu.sync_copy(ones_hbm, onesv)
        pltpu.sync_copy(zeros_hbm.at[pl.ds(s * _RPS, _RPS)],
                        acc.at[pl.ds(s * _RPS, _RPS)])
        plsc.subcore_barrier()

        @pl.loop(0, nblk)
        def _(j):
            pltpu.sync_copy(onesv, acc.at[dstv.at[j]], add=True)

        plsc.subcore_barrier()
        pltpu.sync_copy(acc.at[pl.ds(s * _RPS, _RPS)],
                        out_hbm.at[pl.ds(c * _NP + s * _RPS, _RPS)])

    return deg_kernel


def _make_agg_kernel():
    mesh = plsc.VectorSubcoreMesh(core_axis_name="c", subcore_axis_name="s")
    nblk = _EBLK // _NS

    @functools.partial(
        pl.kernel,
        out_type=jax.ShapeDtypeStruct((_NC * _NP, _DH), jnp.float32),
        mesh=mesh,
        scratch_types=[
            pltpu.VMEM((nblk, _B), jnp.int32),
            pltpu.VMEM((nblk, _B), jnp.int32),
            pltpu.VMEM((_K, _B, _DH), jnp.float32),
            pltpu.SemaphoreType.DMA((_K,)),
            pltpu.SemaphoreType.DMA((_K,)),
            pltpu.VMEM_SHARED((_NP, _DH), jnp.float32),
        ],
        compiler_params=pltpu.CompilerParams(use_tc_tiling_on_sc=False),
    )
    def agg_kernel(v_hbm, src_hbm, dst_hbm, out_hbm, srcv, dstv, rows,
                   gsem, ssem, acc):
        c = lax.axis_index("c")
        s = lax.axis_index("s")
        pltpu.sync_copy(src_hbm.at[pl.ds(c * _EBLK + s * nblk, nblk)], srcv)
        pltpu.sync_copy(dst_hbm.at[pl.ds(s * nblk, nblk)], dstv)
        pltpu.sync_copy(v_hbm.at[pl.ds(c * _NP + s * _RPS, _RPS)],
                        acc.at[pl.ds(s * _RPS, _RPS)])
        plsc.subcore_barrier()

        for b in range(_K):
            pltpu.async_copy(v_hbm.at[srcv.at[b]], rows.at[b], gsem.at[b])

        @pl.loop(0, nblk, step=_K)
        def _(j):
            for b in range(_K):
                pltpu.make_async_copy(v_hbm.at[srcv.at[0]], rows.at[b],
                                      gsem.at[b]).wait()
                pltpu.async_copy(rows.at[b], acc.at[dstv.at[j + b]],
                                 ssem.at[b], add=True)
            for b in range(_K):
                pltpu.make_async_copy(rows.at[b], acc.at[dstv.at[0]],
                                      ssem.at[b]).wait()

                @pl.when(j + _K < nblk)
                def _():
                    pltpu.async_copy(v_hbm.at[srcv.at[j + _K + b]],
                                     rows.at[b], gsem.at[b])

        plsc.subcore_barrier()
        pltpu.sync_copy(acc.at[pl.ds(s * _RPS, _RPS)],
                        out_hbm.at[pl.ds(c * _NP + s * _RPS, _RPS)])

    return agg_kernel


_deg = _make_deg_kernel()
_agg = _make_agg_kernel()


def _scale0_body(x_ref, cnt_ref, xs_ref, d_ref):
    c0 = cnt_ref[0:_N, 0:1]
    c1 = cnt_ref[_NP:_NP + _N, 0:1]
    d = lax.rsqrt(1.0 + c0 + c1)
    d_ref[...] = d
    xs = x_ref[...] * d
    xs_ref[0:_N, :] = xs[:, 0:64]
    xs_ref[_NP:_NP + _N, :] = xs[:, 64:128]


def _scale0(x, cnt):
    return pl.pallas_call(
        _scale0_body,
        out_shape=(jax.ShapeDtypeStruct((2 * _NP, _DH), jnp.float32),
                   jax.ShapeDtypeStruct((_N, 1), jnp.float32)),
    )(x, cnt)


def _mid_body(a_ref, d_ref, b1_ref, w1_ref, w2_ref, o_ref):
    d = d_ref[...]
    aggx = jnp.concatenate([a_ref[0:_N, :], a_ref[_NP:_NP + _N, :]], axis=1)
    h1 = jnp.maximum(
        jnp.dot(aggx * d, w1_ref[...], preferred_element_type=jnp.float32)
        + b1_ref[...], 0.0)
    g = jnp.dot(h1, w2_ref[...], preferred_element_type=jnp.float32) * d
    o_ref[0:_N, :] = g[:, 0:64]
    o_ref[_NP:_NP + _N, :] = g[:, 64:128]


def _mid(a1, d, b1, w1, w2):
    return pl.pallas_call(
        _mid_body,
        out_shape=jax.ShapeDtypeStruct((2 * _NP, _DH), jnp.float32),
    )(a1, d, b1, w1, w2)


def _fin_body(a_ref, d_ref, b2_ref, o_ref):
    d = d_ref[...]
    lo = a_ref[0:_N, :]
    hi = a_ref[_NP:_NP + _N, :]
    o_ref[...] = jnp.concatenate([lo, hi], axis=1) * d + b2_ref[...]


def _fin(acc2, d, b2):
    return pl.pallas_call(
        _fin_body,
        out_shape=jax.ShapeDtypeStruct((_N, 128), jnp.float32),
    )(acc2, d, b2)


def kernel(x, edge_index, W1, b1, W2, b2):
    src = edge_index[0]
    dst = edge_index[1]
    npad = _EP - _E
    srcp = jnp.concatenate([src, jnp.zeros((npad,), jnp.int32)])
    dstp = jnp.concatenate([dst, jnp.full((npad,), _N, jnp.int32)])
    srcp, dstp = lax.sort_key_val(srcp, dstp)
    dst2 = dstp.reshape(_EBLK, _B)
    src2 = jnp.concatenate([srcp, srcp + _NP]).reshape(2 * _EBLK, _B)
    ones16 = jnp.ones((_B, 16), jnp.float32)
    zeros16 = jnp.zeros((_NP, 16), jnp.float32)

    cnt = _deg(dst2, ones16, zeros16)
    xs, d = _scale0(x, cnt)
    a1 = _agg(xs, src2, dst2)
    g = _mid(a1, d, b1.reshape(1, -1), W1, W2)
    a2 = _agg(g, src2, dst2)
    return _fin(a2, d, b2.reshape(1, -1))


# speedup vs baseline: 1.5371x; 1.5371x over previous
import functools

import jax
import jax.numpy as jnp
from jax import lax
from jax.experimental import pallas as pl
from jax.experimental.pallas import tpu as pltpu
from jax.experimental.pallas import tpu_sc as plsc

_N = 10000
_E = 320000
_NC = 2
_NS = 16
_B = 128
_NP = 10112
_RPS = _NP // _NS
_EP = 327680
_EBLK = _EP // _B
_DH = 64
_DW = _DH // 2
_K = 4


def _make_deg_kernel():
    mesh = plsc.VectorSubcoreMesh(core_axis_name="c", subcore_axis_name="s")
    nblk = _EBLK // (_NC * _NS)

    @functools.partial(
        pl.kernel,
        out_type=jax.ShapeDtypeStruct((_NC * _NP, 16), jnp.float32),
        mesh=mesh,
        scratch_types=[
            pltpu.VMEM((nblk, _B), jnp.int32),
            pltpu.VMEM((_B, 16), jnp.float32),
            pltpu.VMEM_SHARED((_NP, 16), jnp.float32),
        ],
        compiler_params=pltpu.CompilerParams(use_tc_tiling_on_sc=False),
    )
    def deg_kernel(dst_hbm, ones_hbm, zeros_hbm, out_hbm, dstv, onesv, acc):
        c = lax.axis_index("c")
        s = lax.axis_index("s")
        w = c * _NS + s
        pltpu.sync_copy(dst_hbm.at[pl.ds(w * nblk, nblk)], dstv)
        pltpu.sync_copy(ones_hbm, onesv)
        pltpu.sync_copy(zeros_hbm.at[pl.ds(s * _RPS, _RPS)],
                        acc.at[pl.ds(s * _RPS, _RPS)])
        plsc.subcore_barrier()

        @pl.loop(0, nblk)
        def _(j):
            pltpu.sync_copy(onesv, acc.at[dstv.at[j]], add=True)

        plsc.subcore_barrier()
        pltpu.sync_copy(acc.at[pl.ds(s * _RPS, _RPS)],
                        out_hbm.at[pl.ds(c * _NP + s * _RPS, _RPS)])

    return deg_kernel


def _make_agg_kernel():
    mesh = plsc.VectorSubcoreMesh(core_axis_name="c", subcore_axis_name="s")
    nblk = _EBLK // _NS
    himask = jnp.int32(-65536)

    @functools.partial(
        pl.kernel,
        out_type=jax.ShapeDtypeStruct((_NC * _NP, _DH), jnp.float32),
        mesh=mesh,
        scratch_types=[
            pltpu.VMEM((nblk, _B), jnp.int32),
            pltpu.VMEM((nblk, _B), jnp.int32),
            pltpu.VMEM((_K, _B, _DW), jnp.int32),
            pltpu.VMEM((_K, _B, _DH), jnp.float32),
            pltpu.SemaphoreType.DMA((_K,)),
            pltpu.SemaphoreType.DMA((_K,)),
            pltpu.VMEM_SHARED((_NP, _DH), jnp.float32),
        ],
        compiler_params=pltpu.CompilerParams(use_tc_tiling_on_sc=False,
                                             needs_layout_passes=False),
    )
    def agg_kernel(vb_hbm, vf_hbm, src_hbm, dst_hbm, out_hbm, srcv, dstv,
                   braw, rows, gsem, ssem, acc):
        c = lax.axis_index("c")
        s = lax.axis_index("s")
        pltpu.sync_copy(src_hbm.at[pl.ds(c * _EBLK + s * nblk, nblk)], srcv)
        pltpu.sync_copy(dst_hbm.at[pl.ds(s * nblk, nblk)], dstv)
        pltpu.sync_copy(vf_hbm.at[pl.ds(c * _NP + s * _RPS, _RPS)],
                        acc.at[pl.ds(s * _RPS, _RPS)])
        plsc.subcore_barrier()

        def widen(b):
            @pl.loop(0, _B, step=4)
            def _(r0):
                for dr in range(4):
                    r = r0 + dr
                    for g in range(2):
                        v = braw.at[b].at[r][pl.ds(g * 16, 16)]
                        even = plsc.bitcast(lax.shift_left(v, 16),
                                            jnp.float32)
                        odd = plsc.bitcast(v & himask, jnp.float32)
                        rows.at[b].at[r][pl.ds(g * 32, 16)] = even
                        rows.at[b].at[r][pl.ds(g * 32 + 16, 16)] = odd

        for b in range(_K):
            pltpu.async_copy(vb_hbm.at[srcv.at[b]], braw.at[b], gsem.at[b])

        @pl.loop(0, nblk, step=_K)
        def _(j):
            for b in range(_K):
                pltpu.make_async_copy(vb_hbm.at[srcv.at[0]], braw.at[b],
                                      gsem.at[b]).wait()
                widen(b)
                pltpu.async_copy(rows.at[b], acc.at[dstv.at[j + b]],
                                 ssem.at[b], add=True)
            for b in range(_K):
                pltpu.make_async_copy(rows.at[b], acc.at[dstv.at[0]],
                                      ssem.at[b]).wait()

                @pl.when(j + _K < nblk)
                def _():
                    pltpu.async_copy(vb_hbm.at[srcv.at[j + _K + b]],
                                     braw.at[b], gsem.at[b])

        plsc.subcore_barrier()
        pltpu.sync_copy(acc.at[pl.ds(s * _RPS, _RPS)],
                        out_hbm.at[pl.ds(c * _NP + s * _RPS, _RPS)])

    return agg_kernel


_deg = _make_deg_kernel()
_agg = _make_agg_kernel()


def _rtne_bf16_bits(x):
    b = jax.lax.bitcast_convert_type(x, jnp.int32)
    return b + jnp.int32(0x7FFF) + (lax.shift_right_logical(b, 16) & 1)


def _pack_rows(v):
    tt = v.reshape(-1, 2, 2, 16)
    a = tt[:, :, 0, :].reshape(-1, 32)
    b = tt[:, :, 1, :].reshape(-1, 32)
    lo = lax.shift_right_logical(_rtne_bf16_bits(a), 16)
    hi = _rtne_bf16_bits(b) & jnp.int32(-65536)
    return lo | hi


_R = 2000


def _scale0_body(x_ref, cnt_ref, xsb_ref, xsf_ref, d_ref):
    c0 = cnt_ref[0, :, 0:1]
    c1 = cnt_ref[1, :, 0:1]
    d = lax.rsqrt(1.0 + c0 + c1)
    d_ref[...] = d
    xs = x_ref[...] * d
    lo = xs[:, 0:64]
    hi = xs[:, 64:128]
    xsf_ref[0] = lo
    xsf_ref[1] = hi
    xsb_ref[0] = _pack_rows(lo)
    xsb_ref[1] = _pack_rows(hi)


def _scale0(x, cnt3):
    return pl.pallas_call(
        _scale0_body,
        grid=(_N // _R,),
        in_specs=[pl.BlockSpec((_R, 128), lambda i: (i, 0)),
                  pl.BlockSpec((2, _R, 16), lambda i: (0, i, 0))],
        out_specs=(pl.BlockSpec((2, _R, _DW), lambda i: (0, i, 0)),
                   pl.BlockSpec((2, _R, _DH), lambda i: (0, i, 0)),
                   pl.BlockSpec((_R, 1), lambda i: (i, 0))),
        out_shape=(jax.ShapeDtypeStruct((2, _NP, _DW), jnp.int32),
                   jax.ShapeDtypeStruct((2, _NP, _DH), jnp.float32),
                   jax.ShapeDtypeStruct((_N, 1), jnp.float32)),
    )(x, cnt3)


def _mid_body(a_ref, d_ref, b1_ref, w1_ref, w2_ref, gb_ref, gf_ref):
    d = d_ref[...]
    aggx = jnp.concatenate([a_ref[0], a_ref[1]], axis=1)
    h1 = jnp.maximum(
        jnp.dot(aggx * d, w1_ref[...], preferred_element_type=jnp.float32)
        + b1_ref[...], 0.0)
    g = jnp.dot(h1, w2_ref[...], preferred_element_type=jnp.float32) * d
    lo = g[:, 0:64]
    hi = g[:, 64:128]
    gf_ref[0] = lo
    gf_ref[1] = hi
    gb_ref[0] = _pack_rows(lo)
    gb_ref[1] = _pack_rows(hi)


def _mid(a13, d, b1, w1, w2):
    return pl.pallas_call(
        _mid_body,
        grid=(_N // _R,),
        in_specs=[pl.BlockSpec((2, _R, _DH), lambda i: (0, i, 0)),
                  pl.BlockSpec((_R, 1), lambda i: (i, 0)),
                  pl.BlockSpec((1, 256), lambda i: (0, 0)),
                  pl.BlockSpec((128, 256), lambda i: (0, 0)),
                  pl.BlockSpec((256, 128), lambda i: (0, 0))],
        out_specs=(pl.BlockSpec((2, _R, _DW), lambda i: (0, i, 0)),
                   pl.BlockSpec((2, _R, _DH), lambda i: (0, i, 0))),
        out_shape=(jax.ShapeDtypeStruct((2, _NP, _DW), jnp.int32),
                   jax.ShapeDtypeStruct((2, _NP, _DH), jnp.float32)),
    )(a13, d, b1, w1, w2)


def _fin_body(a_ref, d_ref, b2_ref, o_ref):
    d = d_ref[...]
    o_ref[...] = jnp.concatenate([a_ref[0], a_ref[1]], axis=1) * d + b2_ref[...]


def _fin(a23, d, b2):
    return pl.pallas_call(
        _fin_body,
        grid=(_N // _R,),
        in_specs=[pl.BlockSpec((2, _R, _DH), lambda i: (0, i, 0)),
                  pl.BlockSpec((_R, 1), lambda i: (i, 0)),
                  pl.BlockSpec((1, 128), lambda i: (0, 0))],
        out_specs=pl.BlockSpec((_R, 128), lambda i: (i, 0)),
        out_shape=jax.ShapeDtypeStruct((_N, 128), jnp.float32),
    )(a23, d, b2)


def kernel(x, edge_index, W1, b1, W2, b2):
    src = edge_index[0]
    dst = edge_index[1]
    npad = _EP - _E
    srcp = jnp.concatenate([src, jnp.zeros((npad,), jnp.int32)])
    dstp = jnp.concatenate([dst, jnp.full((npad,), _N, jnp.int32)])
    dst2 = dstp.reshape(_EBLK, _B)
    src2 = jnp.concatenate([srcp, srcp + _NP]).reshape(2 * _EBLK, _B)
    ones16 = jnp.ones((_B, 16), jnp.float32)
    zeros16 = jnp.zeros((_NP, 16), jnp.float32)

    cnt = _deg(dst2, ones16, zeros16)
    xsb3, xsf3, d = _scale0(x, cnt.reshape(2, _NP, 16))
    a1 = _agg(xsb3.reshape(2 * _NP, _DW), xsf3.reshape(2 * _NP, _DH),
              src2, dst2)
    gb3, gf3 = _mid(a1.reshape(2, _NP, _DH), d, b1.reshape(1, -1), W1, W2)
    a2 = _agg(gb3.reshape(2 * _NP, _DW), gf3.reshape(2 * _NP, _DH),
              src2, dst2)
    return _fin(a2.reshape(2, _NP, _DH), d, b2.reshape(1, -1))


# speedup vs baseline: 1.7186x; 1.1180x over previous
import functools

import jax
import jax.numpy as jnp
from jax import lax
from jax.experimental import pallas as pl
from jax.experimental.pallas import tpu as pltpu
from jax.experimental.pallas import tpu_sc as plsc

_N = 10000
_E = 320000
_NC = 2
_NS = 16
_B = 128
_NP = 10112
_RPS = _NP // _NS
_EP = 327680
_EBLK = _EP // _B
_DH = 64
_K = 5


def _make_deg_kernel():
    mesh = plsc.VectorSubcoreMesh(core_axis_name="c", subcore_axis_name="s")
    nblk = _EBLK // (_NC * _NS)

    @functools.partial(
        pl.kernel,
        out_type=jax.ShapeDtypeStruct((_NC * _NP, 16), jnp.float32),
        mesh=mesh,
        scratch_types=[
            pltpu.VMEM((nblk, _B), jnp.int32),
            pltpu.VMEM((_B, 16), jnp.float32),
            pltpu.VMEM_SHARED((_NP, 16), jnp.float32),
        ],
        compiler_params=pltpu.CompilerParams(use_tc_tiling_on_sc=False),
    )
    def deg_kernel(dst_hbm, ones_hbm, zeros_hbm, out_hbm, dstv, onesv, acc):
        c = lax.axis_index("c")
        s = lax.axis_index("s")
        w = c * _NS + s
        pltpu.sync_copy(dst_hbm.at[pl.ds(w * nblk, nblk)], dstv)
        pltpu.sync_copy(ones_hbm, onesv)
        pltpu.sync_copy(zeros_hbm.at[pl.ds(s * _RPS, _RPS)],
                        acc.at[pl.ds(s * _RPS, _RPS)])
        plsc.subcore_barrier()

        @pl.loop(0, nblk)
        def _(j):
            pltpu.sync_copy(onesv, acc.at[dstv.at[j]], add=True)

        plsc.subcore_barrier()
        pltpu.sync_copy(acc.at[pl.ds(s * _RPS, _RPS)],
                        out_hbm.at[pl.ds(c * _NP + s * _RPS, _RPS)])

    return deg_kernel


def _make_agg_kernel():
    mesh = plsc.VectorSubcoreMesh(core_axis_name="c", subcore_axis_name="s")
    nblk = _EBLK // _NS

    @functools.partial(
        pl.kernel,
        out_type=jax.ShapeDtypeStruct((_NC * _NP, _DH), jnp.float32),
        mesh=mesh,
        scratch_types=[
            pltpu.VMEM((nblk, _B), jnp.int32),
            pltpu.VMEM((nblk, _B), jnp.int32),
            pltpu.VMEM((_K, _B, _DH), jnp.float32),
            pltpu.SemaphoreType.DMA((_K,)),
            pltpu.SemaphoreType.DMA((_K,)),
            pltpu.VMEM_SHARED((_NP, _DH), jnp.float32),
        ],
        compiler_params=pltpu.CompilerParams(use_tc_tiling_on_sc=False),
    )
    def agg_kernel(v_hbm, src_hbm, dst_hbm, out_hbm, srcv, dstv, rows,
                   gsem, ssem, acc):
        c = lax.axis_index("c")
        s = lax.axis_index("s")
        cp_src = pltpu.async_copy(
            src_hbm.at[pl.ds(c * _EBLK + s * nblk, nblk)], srcv, gsem.at[0])
        cp_dst = pltpu.async_copy(
            dst_hbm.at[pl.ds(s * nblk, nblk)], dstv, gsem.at[1])
        cp_ini = pltpu.async_copy(
            v_hbm.at[pl.ds(c * _NP + s * _RPS, _RPS)],
            acc.at[pl.ds(s * _RPS, _RPS)], ssem.at[0])
        cp_src.wait()
        cp_dst.wait()
        cp_ini.wait()
        plsc.subcore_barrier()

        for b in range(_K):
            pltpu.async_copy(v_hbm.at[srcv.at[b]], rows.at[b], gsem.at[b])

        @pl.loop(0, nblk, step=_K)
        def _(j):
            for b in range(_K):
                pltpu.make_async_copy(v_hbm.at[srcv.at[0]], rows.at[b],
                                      gsem.at[b]).wait()
                pltpu.async_copy(rows.at[b], acc.at[dstv.at[j + b]],
                                 ssem.at[b], add=True)
            for b in range(_K):
                pltpu.make_async_copy(rows.at[b], acc.at[dstv.at[0]],
                                      ssem.at[b]).wait()

                @pl.when(j + _K < nblk)
                def _():
                    pltpu.async_copy(v_hbm.at[srcv.at[j + _K + b]],
                                     rows.at[b], gsem.at[b])

        plsc.subcore_barrier()
        pltpu.sync_copy(acc.at[pl.ds(s * _RPS, _RPS)],
                        out_hbm.at[pl.ds(c * _NP + s * _RPS, _RPS)])

    return agg_kernel


_deg = _make_deg_kernel()
_agg = _make_agg_kernel()


def _scale0_body(x_ref, cnt_ref, xs_ref, d_ref):
    c0 = cnt_ref[0:_N, 0:1]
    c1 = cnt_ref[_NP:_NP + _N, 0:1]
    d = lax.rsqrt(1.0 + c0 + c1)
    d_ref[...] = d
    xs = x_ref[...] * d
    xs_ref[0:_N, :] = xs[:, 0:64]
    xs_ref[_NP:_NP + _N, :] = xs[:, 64:128]


def _scale0(x, cnt):
    return pl.pallas_call(
        _scale0_body,
        out_shape=(jax.ShapeDtypeStruct((2 * _NP, _DH), jnp.float32),
                   jax.ShapeDtypeStruct((_N, 1), jnp.float32)),
    )(x, cnt)


def _mid_body(a_ref, d_ref, b1_ref, w1_ref, w2_ref, o_ref):
    d = d_ref[...]
    aggx = jnp.concatenate([a_ref[0:_N, :], a_ref[_NP:_NP + _N, :]], axis=1)
    h1 = jnp.maximum(
        jnp.dot(aggx * d, w1_ref[...], preferred_element_type=jnp.float32)
        + b1_ref[...], 0.0)
    g = jnp.dot(h1, w2_ref[...], preferred_element_type=jnp.float32) * d
    o_ref[0:_N, :] = g[:, 0:64]
    o_ref[_NP:_NP + _N, :] = g[:, 64:128]


def _mid(a1, d, b1, w1, w2):
    return pl.pallas_call(
        _mid_body,
        out_shape=jax.ShapeDtypeStruct((2 * _NP, _DH), jnp.float32),
    )(a1, d, b1, w1, w2)


def _fin_body(a_ref, d_ref, b2_ref, o_ref):
    d = d_ref[...]
    lo = a_ref[0:_N, :]
    hi = a_ref[_NP:_NP + _N, :]
    o_ref[...] = jnp.concatenate([lo, hi], axis=1) * d + b2_ref[...]


def _fin(acc2, d, b2):
    return pl.pallas_call(
        _fin_body,
        out_shape=jax.ShapeDtypeStruct((_N, 128), jnp.float32),
    )(acc2, d, b2)


def kernel(x, edge_index, W1, b1, W2, b2):
    src = edge_index[0]
    dst = edge_index[1]
    npad = _EP - _E
    srcp = jnp.concatenate([src, jnp.zeros((npad,), jnp.int32)])
    dstp = jnp.concatenate([dst, jnp.full((npad,), _N, jnp.int32)])
    dst2 = dstp.reshape(_EBLK, _B)
    src2 = jnp.concatenate([srcp, srcp + _NP]).reshape(2 * _EBLK, _B)
    ones16 = jnp.ones((_B, 16), jnp.float32)
    zeros16 = jnp.zeros((_NP, 16), jnp.float32)

    cnt = _deg(dst2, ones16, zeros16)
    xs, d = _scale0(x, cnt)
    a1 = _agg(xs, src2, dst2)
    g = _mid(a1, d, b1.reshape(1, -1), W1, W2)
    a2 = _agg(g, src2, dst2)
    return _fin(a2, d, b2.reshape(1, -1))
